# baseline (speedup 1.0000x reference)
import functools

import jax
import jax.numpy as jnp
from jax import lax
from jax.experimental import pallas as pl
from jax.experimental.pallas import tpu as pltpu
from jax.experimental.pallas import tpu_sc as plsc

B = 4
L = 8192
D = 64
N_IDX = B * L

_info = plsc.get_sparse_core_info()
NC, NS = _info.num_cores, _info.num_subcores
NW = NC * NS
B_W = N_IDX // NW
CH = 32
NCH = B_W // CH
NPAIR = NCH // 2

_mesh = plsc.VectorSubcoreMesh(core_axis_name="c", subcore_axis_name="s")


@functools.partial(
    pl.kernel,
    mesh=_mesh,
    compiler_params=pltpu.CompilerParams(use_tc_tiling_on_sc=True),
    out_type=jax.ShapeDtypeStruct((N_IDX, D), jnp.float32),
    scratch_types=[
        pltpu.VMEM((B_W,), jnp.int32),
        pltpu.VMEM((2, CH, 512), jnp.float32),
        pltpu.VMEM((2, CH, D), jnp.float32),
        pltpu.SemaphoreType.DMA,
        pltpu.SemaphoreType.DMA,
        pltpu.SemaphoreType.DMA,
        pltpu.SemaphoreType.DMA,
    ],
)
def _embed_gather(idx_hbm, table8_hbm, out_hbm, idx_v, buf_v, sel_v,
                  g_sem0, g_sem1, o_sem0, o_sem1):
    wid = lax.axis_index("s") * NC + lax.axis_index("c")
    base = wid * B_W
    b = wid // (L // B_W)
    l0 = (wid % (L // B_W)) * B_W
    pltpu.sync_copy(idx_hbm.at[b, pl.ds(l0, B_W)], idx_v)

    g_sems = (g_sem0, g_sem1)
    o_sems = (o_sem0, o_sem1)

    def gather(ct, slot):
        def issue_group(g, _):
            v16 = idx_v[pl.ds(ct * CH + g * 16, 16)]
            for l in range(16):
                pltpu.make_async_copy(
                    table8_hbm.at[pl.ds(lax.shift_right_logical(v16[l], 3), 1)],
                    buf_v.at[slot, pl.ds(g * 16 + l, 1)],
                    g_sems[slot],
                ).start()
            return _

        lax.fori_loop(0, CH // 16, issue_group, 0)

    def gather_wait(slot):
        def wait_group(g, _):
            for l in range(16):
                pltpu.make_async_copy(
                    table8_hbm.at[pl.ds(0, 1)],
                    buf_v.at[slot, pl.ds(0, 1)],
                    g_sems[slot],
                ).wait()
            return _

        lax.fori_loop(0, CH // 16, wait_group, 0)

    def select(ct, slot):
        def sel_group(g, _):
            s16 = (idx_v[pl.ds(ct * CH + g * 16, 16)] & 7) * D
            for l in range(16):
                i = g * 16 + l
                off = pl.multiple_of(s16[l], D)
                for c0 in range(0, D, 16):
                    sel_v[slot, i, pl.ds(c0, 16)] = (
                        buf_v[slot, i, pl.ds(off + c0, 16)]
                    )
            return _

        lax.fori_loop(0, CH // 16, sel_group, 0)

    def out_start(ct, slot):
        pltpu.make_async_copy(
            sel_v.at[slot],
            out_hbm.at[pl.ds(base + ct * CH, CH)],
            o_sems[slot],
        ).start()

    def out_wait(slot):
        pltpu.make_async_copy(
            sel_v.at[slot],
            out_hbm.at[pl.ds(base, CH)],
            o_sems[slot],
        ).wait()

    gather(0, 0)
    gather(1, 1)
    gather_wait(0)
    select(0, 0)
    out_start(0, 0)
    gather(2, 0)
    gather_wait(1)
    select(1, 1)
    out_start(1, 1)
    gather(3, 1)

    def pair(t, _):
        c0_ = 2 * t
        c1_ = 2 * t + 1
        gather_wait(0)
        out_wait(0)
        select(c0_, 0)
        out_start(c0_, 0)
        gather(jnp.minimum(c0_ + 2, NCH - 2), 0)
        gather_wait(1)
        out_wait(1)
        select(c1_, 1)
        out_start(c1_, 1)
        gather(jnp.minimum(c1_ + 2, NCH - 1), 1)
        return _

    lax.fori_loop(1, NPAIR, pair, 0)

    out_wait(0)
    out_wait(1)
    gather_wait(0)
    gather_wait(1)


def kernel(x_BL, table):
    table8 = table.reshape(125000, 512)
    out = _embed_gather(x_BL.astype(jnp.int32), table8)
    return out.reshape(B, L, D)


# speedup vs baseline: 1.8015x; 1.8015x over previous
import functools

import jax
import jax.numpy as jnp
from jax import lax
from jax.experimental import pallas as pl
from jax.experimental.pallas import tpu as pltpu
from jax.experimental.pallas import tpu_sc as plsc

B = 4
L = 8192
D = 64
N_IDX = B * L

_info = plsc.get_sparse_core_info()
NC, NS = _info.num_cores, _info.num_subcores
NW = NC * NS
B_W = N_IDX // NW
CH = 128
NCH = B_W // CH
NG = CH // 16

_mesh = plsc.VectorSubcoreMesh(core_axis_name="c", subcore_axis_name="s")


@functools.partial(
    pl.kernel,
    mesh=_mesh,
    compiler_params=pltpu.CompilerParams(use_tc_tiling_on_sc=True),
    out_type=jax.ShapeDtypeStruct((N_IDX, D), jnp.float32),
    scratch_types=[
        pltpu.VMEM((B_W,), jnp.int32),
        pltpu.VMEM((3, CH, D), jnp.float32),
        pltpu.SemaphoreType.DMA,
        pltpu.SemaphoreType.DMA,
        pltpu.SemaphoreType.DMA,
        pltpu.SemaphoreType.DMA,
    ],
)
def _embed_gather(idx_hbm, table_hbm, out_hbm, idx_v, buf_v,
                  g_sem, o_sem0, o_sem1, o_sem2):
    wid = lax.axis_index("s") * NC + lax.axis_index("c")
    base = wid * B_W
    b = wid // (L // B_W)
    l0 = (wid % (L // B_W)) * B_W
    pltpu.sync_copy(idx_hbm.at[b, pl.ds(l0, B_W)], idx_v)

    o_sems = (o_sem0, o_sem1, o_sem2)

    def issue_chunk(j):
        jb = j % 3

        def issue_group(g, _):
            v16 = idx_v[pl.ds(j * CH + g * 16, 16)]
            for l in range(16):
                pltpu.make_async_copy(
                    table_hbm.at[pl.ds(v16[l], 1)],
                    buf_v.at[jb, pl.ds(g * 16 + l, 1)],
                    g_sem,
                ).start()
            return _

        lax.fori_loop(0, NG, issue_group, 0)

    def wait_chunk(j):
        def wait_group(g, _):
            for l in range(16):
                pltpu.make_async_copy(
                    table_hbm.at[pl.ds(0, 1)],
                    buf_v.at[0, pl.ds(0, 1)],
                    g_sem,
                ).wait()
            return _

        lax.fori_loop(0, NG, wait_group, 0)

    def out_copy(j):
        return pltpu.make_async_copy(
            buf_v.at[j % 3],
            out_hbm.at[pl.ds(base + j * CH, CH)],
            o_sems[j % 3],
        )

    issue_chunk(0)
    for j in range(NCH):
        if j + 1 < NCH:
            if j >= 2:
                out_copy(j - 2).wait()
            issue_chunk(j + 1)
        wait_chunk(j)
        out_copy(j).start()

    out_copy(NCH - 3).wait()
    out_copy(NCH - 2).wait()
    out_copy(NCH - 1).wait()


def kernel(x_BL, table):
    out = _embed_gather(x_BL.astype(jnp.int32), table)
    return out.reshape(B, L, D)
